# Initial kernel scaffold; baseline (speedup 1.0000x reference)
import jax
import jax.numpy as jnp
from jax.experimental import pallas as pl


def kernel(x, node_ids, neigh1, alpha1, neigh2, alpha2, W_self1, b_self1, W_agg1, b_agg1, W_out1, b_out1, W_self0, b_self0, W_agg0, b_agg0, W_out0, b_out0, W_emb, b_emb):
    raise NotImplementedError("write your pallas kernel here")


# speedup vs baseline: 3.5728x; 3.5728x over previous
import functools

import jax
import jax.numpy as jnp
from jax import lax
from jax.experimental import pallas as pl
from jax.experimental.pallas import tpu as pltpu
from jax.experimental.pallas import tpu_sc as plsc

NN = 100000
DD = 128
NB = 4096
NT = 10
M1 = NB * (1 + NT)

NC, NS = 2, 16
NW = NC * NS
RPW = M1 // NW
CH = 32
NCHUNK = RPW // CH
TPAD = 16


def _t1_body(x_ref, ws, bs, wa, ba, wot, wob, sp_ref, ap_ref):
    xb = x_ref[...]
    s = jnp.maximum(jnp.dot(xb, ws[...], preferred_element_type=jnp.float32) + bs[...], 0.0)
    sp_ref[...] = jnp.dot(s, wot[...], preferred_element_type=jnp.float32)
    a = jnp.maximum(jnp.dot(xb, wa[...], preferred_element_type=jnp.float32) + ba[...], 0.0)
    ap_ref[...] = jnp.dot(a, wob[...], preferred_element_type=jnp.float32)


_T1_BM = 2000


def _t1(x, ws, bs, wa, ba, wot, wob):
    full = pl.BlockSpec((DD, DD), lambda i: (0, 0))
    row = pl.BlockSpec((1, DD), lambda i: (0, 0))
    blk = pl.BlockSpec((_T1_BM, DD), lambda i: (i, 0))
    return pl.pallas_call(
        _t1_body,
        grid=(NN // _T1_BM,),
        in_specs=[blk, full, row, full, row, full, full],
        out_specs=[blk, blk],
        out_shape=[jax.ShapeDtypeStruct((NN, DD), jnp.float32)] * 2,
    )(x, ws, bs, wa, ba, wot, wob)


def _sc_body(sp_hbm, ap_hbm, idxs_hbm, idxn_hbm, al_hbm, b_hbm,
             e1t_hbm, e1n_hbm, sq_hbm,
             idxs_v, idxn_v, al_v, self_v, nbr_v, out_v, b_v, sq_v,
             sem_s, sem_n):
    wid = lax.axis_index("s") * NC + lax.axis_index("c")
    base = wid * RPW
    pltpu.sync_copy(b_hbm, b_v)
    sq_v[0, :] = jnp.zeros((16,), jnp.float32)
    sq_v[1, :] = jnp.zeros((16,), jnp.float32)

    def chunk(j, carry):
        r0 = base + j * CH
        pltpu.sync_copy(idxs_hbm.at[pl.ds(r0, CH)], idxs_v)
        pltpu.sync_copy(idxn_hbm.at[pl.ds(r0 * NT, CH * NT)], idxn_v)
        pltpu.sync_copy(al_hbm.at[pl.ds(r0, CH)], al_v)
        cp_s = pltpu.async_copy(sp_hbm.at[idxs_v], self_v, sem_s)
        cp_n = pltpu.async_copy(ap_hbm.at[idxn_v], nbr_v, sem_n)
        cp_s.wait()
        cp_n.wait()

        def row(i, sqc):
            al_row = al_v[i, :]
            for c in range(DD // 16):
                sl = pl.ds(c * 16, 16)
                acc = self_v[i, sl] + b_v[sl]
                for t in range(NT):
                    acc = acc + al_row[t] * nbr_v[i * NT + t, sl]
                r = jnp.maximum(acc, 0.0)
                out_v[i, sl] = r
                sqc = sqc + r * r
            return sqc

        sqc = lax.fori_loop(0, CH, row, jnp.zeros((16,), jnp.float32))
        is_t = r0 < NB

        @pl.when(is_t)
        def _():
            sq_v[0, :] = sq_v[0, :] + sqc
            pltpu.sync_copy(out_v, e1t_hbm.at[pl.ds(r0, CH)])

        @pl.when(jnp.logical_not(is_t))
        def _():
            sq_v[1, :] = sq_v[1, :] + sqc
            pltpu.sync_copy(out_v, e1n_hbm.at[pl.ds(r0 - NB, CH)])

        return carry

    lax.fori_loop(0, NCHUNK, chunk, 0)
    pltpu.sync_copy(sq_v, sq_hbm.at[wid])


_sc_call = functools.partial(
    pl.kernel,
    out_type=(
        jax.ShapeDtypeStruct((NB, DD), jnp.float32),
        jax.ShapeDtypeStruct((NB * NT, DD), jnp.float32),
        jax.ShapeDtypeStruct((NW, 2, 16), jnp.float32),
    ),
    mesh=plsc.VectorSubcoreMesh(core_axis_name="c", subcore_axis_name="s"),
    scratch_types=[
        pltpu.VMEM((CH,), jnp.int32),
        pltpu.VMEM((CH * NT,), jnp.int32),
        pltpu.VMEM((CH, TPAD), jnp.float32),
        pltpu.VMEM((CH, DD), jnp.float32),
        pltpu.VMEM((CH * NT, DD), jnp.float32),
        pltpu.VMEM((CH, DD), jnp.float32),
        pltpu.VMEM((DD,), jnp.float32),
        pltpu.VMEM((2, 16), jnp.float32),
        pltpu.SemaphoreType.DMA,
        pltpu.SemaphoreType.DMA,
    ],
)(_sc_body)


_BM0 = 512


def _t3b_body(e1t_ref, e1n_ref, al_ref, ws0, bs0, wa0, ba0, wo0t, wo0b, bo0,
              nus_ref, e0_ref, sq_ref):
    nu1 = nus_ref[0, 0]
    nu2 = nus_ref[0, 1]
    zt = jnp.maximum(
        jnp.dot(e1t_ref[...], ws0[...], preferred_element_type=jnp.float32)
        + nu1 * bs0[...], 0.0)
    al = al_ref[...]
    agg = jnp.zeros((_BM0, DD), jnp.float32)
    for t in range(NT):
        znt = jnp.maximum(
            jnp.dot(e1n_ref[t], wa0[...], preferred_element_type=jnp.float32)
            + nu2 * ba0[...], 0.0)
        agg = agg + al[:, t:t + 1] * znt
    pre = (jnp.dot(zt, wo0t[...], preferred_element_type=jnp.float32) / nu1
           + jnp.dot(agg, wo0b[...], preferred_element_type=jnp.float32) / nu2
           + bo0[...])
    e0 = jnp.maximum(pre, 0.0)
    e0_ref[...] = e0

    @pl.when(pl.program_id(0) == 0)
    def _():
        sq_ref[...] = jnp.zeros_like(sq_ref)

    sq_ref[...] += jnp.sum(e0 * e0, axis=0, keepdims=True)


def _t3b(e1t, e1n3, al1, ws0, bs0, wa0, ba0, wo0t, wo0b, bo0, nus):
    full = pl.BlockSpec((DD, DD), lambda i: (0, 0))
    row = pl.BlockSpec((1, DD), lambda i: (0, 0))
    return pl.pallas_call(
        _t3b_body,
        grid=(NB // _BM0,),
        in_specs=[
            pl.BlockSpec((_BM0, DD), lambda i: (i, 0)),
            pl.BlockSpec((NT, _BM0, DD), lambda i: (0, i, 0)),
            pl.BlockSpec((_BM0, TPAD), lambda i: (i, 0)),
            full, row, full, row, full, full, row,
            pl.BlockSpec((1, 2), lambda i: (0, 0), memory_space=pltpu.SMEM),
        ],
        out_specs=[
            pl.BlockSpec((_BM0, DD), lambda i: (i, 0)),
            pl.BlockSpec((1, DD), lambda i: (0, 0)),
        ],
        out_shape=[
            jax.ShapeDtypeStruct((NB, DD), jnp.float32),
            jax.ShapeDtypeStruct((1, DD), jnp.float32),
        ],
    )(e1t, e1n3, al1, ws0, bs0, wa0, ba0, wo0t, wo0b, bo0, nus)


def _t3c_body(e0_ref, wemb, bemb, nu_ref, out_ref):
    nu0 = nu_ref[0, 0]
    q = jnp.maximum(
        jnp.dot(e0_ref[...], wemb[...], preferred_element_type=jnp.float32)
        + nu0 * bemb[...], 0.0)
    out_ref[...] = q * (1.0 / nu0)


def _t3c(e0, wemb, bemb, nu0):
    return pl.pallas_call(
        _t3c_body,
        grid=(2,),
        in_specs=[
            pl.BlockSpec((NB // 2, DD), lambda i: (i, 0)),
            pl.BlockSpec((DD, DD), lambda i: (0, 0)),
            pl.BlockSpec((1, DD), lambda i: (0, 0)),
            pl.BlockSpec((1, 1), lambda i: (0, 0), memory_space=pltpu.SMEM),
        ],
        out_specs=pl.BlockSpec((NB // 2, DD), lambda i: (i, 0)),
        out_shape=jax.ShapeDtypeStruct((NB, DD), jnp.float32),
    )(e0, wemb, bemb, nu0)


def kernel(x, node_ids, neigh1, alpha1, neigh2, alpha2,
           W_self1, b_self1, W_agg1, b_agg1, W_out1, b_out1,
           W_self0, b_self0, W_agg0, b_agg0, W_out0, b_out0,
           W_emb, b_emb):
    idx_self = jnp.concatenate([node_ids, neigh1.T.reshape(-1)])
    nbr_n = neigh2.reshape(NB, NT, NT).transpose(1, 0, 2).reshape(NB * NT, NT)
    idx_nbr = jnp.concatenate([neigh1, nbr_n], axis=0).reshape(-1)
    al_n = alpha2.reshape(NB, NT, NT).transpose(1, 0, 2).reshape(NB * NT, NT)
    alpha_cat = jnp.pad(jnp.concatenate([alpha1, al_n], axis=0),
                        ((0, 0), (0, TPAD - NT)))
    alpha1_pad = alpha_cat[:NB]

    Sp, Ap = _t1(x, W_self1, b_self1.reshape(1, DD), W_agg1,
                 b_agg1.reshape(1, DD), W_out1[:DD], W_out1[DD:])

    e1_t, e1_n, sqp = _sc_call(Sp, Ap, idx_self, idx_nbr, alpha_cat, b_out1)
    nu1 = jnp.sqrt(jnp.sum(sqp[:, 0, :]))
    nu2 = jnp.sqrt(jnp.sum(sqp[:, 1, :]))

    e1n3 = e1_n.reshape(NT, NB, DD)
    nus = jnp.stack([nu1, nu2]).reshape(1, 2)
    e0, sq0 = _t3b(e1_t, e1n3, alpha1_pad, W_self0, b_self0.reshape(1, DD),
                   W_agg0, b_agg0.reshape(1, DD), W_out0[:DD], W_out0[DD:],
                   b_out0.reshape(1, DD), nus)
    nu0 = jnp.sqrt(jnp.sum(sq0)).reshape(1, 1)

    return _t3c(e0, W_emb, b_emb.reshape(1, DD), nu0)
